# baseline (speedup 1.0000x reference)
import functools
import jax
import jax.numpy as jnp
from jax import lax
from jax.experimental import pallas as pl
from jax.experimental.pallas import tpu as pltpu
from jax.experimental.pallas import tpu_sc as plsc

NW = 32
CH = 32


def _matmul_body(x_ref, w_ref, b_ref, logits_ref):
    x = x_ref[...].astype(jnp.bfloat16)
    w = w_ref[...].astype(jnp.bfloat16)
    acc = lax.dot_general(x, w, (((1,), (1,)), ((), ())),
                          preferred_element_type=jnp.float32)
    logits_ref[...] = acc + b_ref[...]


def _onehot_body(V, RPW, y_hbm, out_hbm, idx_v, bufA, bufB, semA, semB):
    wid = lax.axis_index("s") * 2 + lax.axis_index("c")
    base = wid * RPW
    pltpu.sync_copy(y_hbm.at[pl.ds(base, RPW)], idx_v)

    def zero_body(i, _):
        z = jnp.zeros((16,), jnp.float32)
        bufA[pl.ds(i * 16, 16)] = z
        bufB[pl.ds(i * 16, 16)] = z
        return 0
    lax.fori_loop(0, CH * V // 16, zero_body, 0)

    ones = jnp.full((16,), 1.0, jnp.float32)
    zeros = jnp.zeros((16,), jnp.float32)
    rows16 = lax.iota(jnp.int32, 16)

    def scatter(buf, c, val):
        for g in range(CH // 16):
            yv = idx_v[pl.ds(c * CH + g * 16, 16)]
            pos = (rows16 + g * 16) * V + yv
            plsc.store_scatter(buf, [pos], val)

    nch = RPW // CH
    copies = [None, None]
    for c in range(nch):
        buf, sem = (bufA, semA) if c % 2 == 0 else (bufB, semB)
        if c >= 2:
            copies[c % 2].wait()
            scatter(buf, c - 2, zeros)
        scatter(buf, c, ones)
        copies[c % 2] = pltpu.async_copy(
            buf, out_hbm.at[pl.ds((base + c * CH) * V, CH * V)], sem)
    copies[0].wait()
    copies[1].wait()


def kernel(outputs, y, W, b, emb):
    del emb
    B, H = outputs.shape
    V = W.shape[0]
    RPW = B // NW

    mesh = plsc.VectorSubcoreMesh(core_axis_name="c", subcore_axis_name="s")
    onehot_flat = pl.kernel(
        functools.partial(_onehot_body, V, RPW),
        mesh=mesh,
        compiler_params=pltpu.CompilerParams(needs_layout_passes=False),
        out_type=jax.ShapeDtypeStruct((B * V,), jnp.float32),
        scratch_types=[
            pltpu.VMEM((RPW,), jnp.int32),
            pltpu.VMEM((CH * V,), jnp.float32),
            pltpu.VMEM((CH * V,), jnp.float32),
            pltpu.SemaphoreType.DMA,
            pltpu.SemaphoreType.DMA,
        ],
    )(y)
    onehot = onehot_flat.reshape(B, V)

    BM = 1024 if B % 1024 == 0 else B
    logits = pl.pallas_call(
        _matmul_body,
        grid=(B // BM,),
        in_specs=[
            pl.BlockSpec((BM, H), lambda i: (i, 0)),
            pl.BlockSpec((V, H), lambda i: (0, 0)),
            pl.BlockSpec((1, V), lambda i: (0, 0)),
        ],
        out_specs=pl.BlockSpec((BM, V), lambda i: (i, 0)),
        out_shape=jax.ShapeDtypeStruct((B, V), jnp.float32),
    )(outputs, W, b.reshape(1, V))
    return (logits, onehot)


# speedup vs baseline: 1.3356x; 1.3356x over previous
import functools
import jax
import jax.numpy as jnp
from jax import lax
from jax.experimental import pallas as pl
from jax.experimental.pallas import tpu as pltpu
from jax.experimental.pallas import tpu_sc as plsc

NW = 32
CH = 32


def _matmul_body(x_ref, w_ref, b_ref, logits_ref):
    x = x_ref[...].astype(jnp.bfloat16)
    w = w_ref[...].astype(jnp.bfloat16)
    acc = lax.dot_general(x, w, (((1,), (1,)), ((), ())),
                          preferred_element_type=jnp.float32)
    logits_ref[...] = acc + b_ref[...]


def _onehot_body(V, RPW, y_hbm, out_hbm, idx_v, bufA, bufB, semA, semB):
    wid = lax.axis_index("s") * 2 + lax.axis_index("c")
    base = wid * RPW
    pltpu.sync_copy(y_hbm.at[pl.ds(base, RPW)], idx_v)

    nz = V // 16
    z = jnp.zeros((16,), jnp.float32)

    def zero_row(r, _):
        def zero_col(cc, _):
            bufA[r, pl.ds(cc * 16, 16)] = z
            bufB[r, pl.ds(cc * 16, 16)] = z
            return 0
        lax.fori_loop(0, nz, zero_col, 0)
        bufA[r, pl.ds(V - 16, 16)] = z
        bufB[r, pl.ds(V - 16, 16)] = z
        return 0
    lax.fori_loop(0, CH, zero_row, 0)

    ones = jnp.full((16,), 1.0, jnp.float32)
    zeros = jnp.zeros((16,), jnp.float32)
    rows16 = lax.iota(jnp.int32, 16)

    def scatter(buf, c, val):
        for g in range(CH // 16):
            yv = idx_v[pl.ds(c * CH + g * 16, 16)]
            plsc.store_scatter(buf, [rows16 + g * 16, yv], val)

    nch = RPW // CH
    copies = [None, None]
    for c in range(nch):
        buf, sem = (bufA, semA) if c % 2 == 0 else (bufB, semB)
        if c >= 2:
            copies[c % 2].wait()
            scatter(buf, c - 2, zeros)
        scatter(buf, c, ones)
        copies[c % 2] = pltpu.async_copy(
            buf, out_hbm.at[pl.ds(base + c * CH, CH)], sem)
    copies[0].wait()
    copies[1].wait()


def kernel(outputs, y, W, b, emb):
    del emb
    B, H = outputs.shape
    V = W.shape[0]
    RPW = B // NW

    mesh = plsc.VectorSubcoreMesh(core_axis_name="c", subcore_axis_name="s")
    onehot = pl.kernel(
        functools.partial(_onehot_body, V, RPW),
        mesh=mesh,
        compiler_params=pltpu.CompilerParams(needs_layout_passes=False),
        out_type=jax.ShapeDtypeStruct((B, V), jnp.float32),
        scratch_types=[
            pltpu.VMEM((RPW,), jnp.int32),
            pltpu.VMEM((CH, V), jnp.float32),
            pltpu.VMEM((CH, V), jnp.float32),
            pltpu.SemaphoreType.DMA,
            pltpu.SemaphoreType.DMA,
        ],
    )(y)

    BM = 1024 if B % 1024 == 0 else B
    logits = pl.pallas_call(
        _matmul_body,
        grid=(B // BM,),
        in_specs=[
            pl.BlockSpec((BM, H), lambda i: (i, 0)),
            pl.BlockSpec((V, H), lambda i: (0, 0)),
            pl.BlockSpec((1, V), lambda i: (0, 0)),
        ],
        out_specs=pl.BlockSpec((BM, V), lambda i: (i, 0)),
        out_shape=jax.ShapeDtypeStruct((B, V), jnp.float32),
    )(outputs, W, b.reshape(1, V))
    return (logits, onehot)


# speedup vs baseline: 3.2153x; 2.4073x over previous
import functools
import jax
import jax.numpy as jnp
from jax import lax
from jax.experimental import pallas as pl
from jax.experimental.pallas import tpu as pltpu
from jax.experimental.pallas import tpu_sc as plsc

NW = 32
CC = 128


def _matmul_body(x_ref, w_ref, b_ref, logits_ref):
    x = x_ref[...].astype(jnp.bfloat16)
    w = w_ref[...].astype(jnp.bfloat16)
    acc = lax.dot_general(w, x, (((1,), (1,)), ((), ())),
                          preferred_element_type=jnp.float32)
    logits_ref[...] = acc + b_ref[...]


def _onehot_body(V, CPW, y_hbm, out_hbm, idx_v, buf, sem):
    wid = lax.axis_index("s") * 2 + lax.axis_index("c")
    base = wid * CPW
    pltpu.sync_copy(y_hbm.at[pl.ds(base, CPW)], idx_v)

    z = jnp.zeros((16,), jnp.float32)

    def zero_row(r, _):
        def zero_col(cc, _):
            buf[r, pl.ds(cc * 16, 16)] = z
            return 0
        lax.fori_loop(0, CC // 16, zero_col, 0)
        return 0
    lax.fori_loop(0, V, zero_row, 0)

    ones = jnp.full((16,), 1.0, jnp.float32)
    zeros = jnp.zeros((16,), jnp.float32)
    cols16 = lax.iota(jnp.int32, 16)

    def scatter(c, val):
        for g in range(CC // 16):
            yv = idx_v[pl.ds(c * CC + g * 16, 16)]
            plsc.store_scatter(buf, [yv, cols16 + g * 16], val)

    nch = CPW // CC
    for c in range(nch):
        scatter(c, ones)
        pltpu.async_copy(
            buf, out_hbm.at[:, pl.ds(base + c * CC, CC)], sem).wait()
        if c + 1 < nch:
            scatter(c, zeros)


def kernel(outputs, y, W, b, emb):
    del emb
    B, H = outputs.shape
    V = W.shape[0]
    CPW = B // NW

    mesh = plsc.VectorSubcoreMesh(core_axis_name="c", subcore_axis_name="s")
    onehot_t = pl.kernel(
        functools.partial(_onehot_body, V, CPW),
        mesh=mesh,
        compiler_params=pltpu.CompilerParams(needs_layout_passes=False),
        out_type=jax.ShapeDtypeStruct((V, B), jnp.float32),
        scratch_types=[
            pltpu.VMEM((CPW,), jnp.int32),
            pltpu.VMEM((V, CC), jnp.float32),
            pltpu.SemaphoreType.DMA,
        ],
    )(y)

    BM = 2048 if B % 2048 == 0 else B
    logits_t = pl.pallas_call(
        _matmul_body,
        grid=(B // BM,),
        in_specs=[
            pl.BlockSpec((BM, H), lambda i: (i, 0)),
            pl.BlockSpec((V, H), lambda i: (0, 0)),
            pl.BlockSpec((V, 1), lambda i: (0, 0)),
        ],
        out_specs=pl.BlockSpec((V, BM), lambda i: (0, i)),
        out_shape=jax.ShapeDtypeStruct((V, B), jnp.float32),
    )(outputs, W, b.reshape(V, 1))
    return (logits_t.T, onehot_t.T)
